# baseline (speedup 1.0000x reference)
import functools

import jax
import jax.numpy as jnp
from jax import lax
from jax.experimental import pallas as pl
from jax.experimental.pallas import tpu as pltpu
from jax.experimental.pallas import tpu_sc as plsc

_B = 16384
_D = 128
_T = 1000

_NC = 2
_NS = 16
_NW = _NC * _NS
_BPW = _B // _NW


def _sc_gather_body(t_hbm, la_hbm, lo_hbm, la_out_hbm, lo_out_hbm,
                    la_tab, lo_tab, idx_v, la_v, lo_v,
                    sem_a, sem_b, sem_c):
    wid = lax.axis_index("s") * _NC + lax.axis_index("c")
    base = wid * _BPW
    cp_a = pltpu.async_copy(la_hbm, la_tab, sem_a)
    cp_b = pltpu.async_copy(lo_hbm, lo_tab, sem_b)
    cp_c = pltpu.async_copy(t_hbm.at[pl.ds(base, _BPW)], idx_v, sem_c)
    cp_a.wait()
    cp_b.wait()
    cp_c.wait()
    for i in range(_BPW // 16):
        sl = pl.ds(i * 16, 16)
        idx = idx_v[sl]
        la_v[sl] = plsc.load_gather(la_tab, [idx])
        lo_v[sl] = plsc.load_gather(lo_tab, [idx])
    cp_d = pltpu.async_copy(la_v, la_out_hbm.at[pl.ds(base, _BPW)], sem_a)
    cp_e = pltpu.async_copy(lo_v, lo_out_hbm.at[pl.ds(base, _BPW)], sem_b)
    cp_d.wait()
    cp_e.wait()


_sc_gather = functools.partial(
    pl.kernel,
    mesh=plsc.VectorSubcoreMesh(core_axis_name="c", subcore_axis_name="s"),
    compiler_params=pltpu.CompilerParams(needs_layout_passes=False),
    out_type=(
        jax.ShapeDtypeStruct((_B,), jnp.float32),
        jax.ShapeDtypeStruct((_B,), jnp.float32),
    ),
    scratch_types=[
        pltpu.VMEM((_T,), jnp.float32),
        pltpu.VMEM((_T,), jnp.float32),
        pltpu.VMEM((_BPW,), jnp.int32),
        pltpu.VMEM((_BPW,), jnp.float32),
        pltpu.VMEM((_BPW,), jnp.float32),
        pltpu.SemaphoreType.DMA,
        pltpu.SemaphoreType.DMA,
        pltpu.SemaphoreType.DMA,
    ],
)(_sc_gather_body)


def _tc_body(x_ref, la_ref, lo_ref, nc_ref, out_ref):
    a = la_ref[...] + x_ref[...]
    b = lo_ref[...] - jnp.log(nc_ref[...])
    m = jnp.maximum(a, b)
    d = jnp.minimum(a, b) - m
    out_ref[...] = m + jnp.log1p(jnp.exp(d))


_ROWS = 4096


def _tc_dense(log_x_0, la_t, lo_t, nc):
    grid = (_B // _ROWS,)
    return pl.pallas_call(
        _tc_body,
        grid=grid,
        in_specs=[
            pl.BlockSpec((_ROWS, _D), lambda i: (i, 0)),
            pl.BlockSpec((_ROWS, 1), lambda i: (i, 0)),
            pl.BlockSpec((_ROWS, 1), lambda i: (i, 0)),
            pl.BlockSpec((1, _D), lambda i: (0, 0)),
        ],
        out_specs=pl.BlockSpec((_ROWS, _D), lambda i: (i, 0)),
        out_shape=jax.ShapeDtypeStruct((_B, _D), jnp.float32),
    )(log_x_0, la_t, lo_t, nc)


def kernel(log_x_0, t, log_alpha_bar, log_one_minus_alpha_bar,
           num_classe_extended):
    la_t, lo_t = _sc_gather(t, log_alpha_bar, log_one_minus_alpha_bar)
    return _tc_dense(
        log_x_0,
        la_t.reshape(_B, 1),
        lo_t.reshape(_B, 1),
        num_classe_extended.reshape(1, _D),
    )


# speedup vs baseline: 1.2766x; 1.2766x over previous
import functools

import jax
import jax.numpy as jnp
from jax import lax
from jax.experimental import pallas as pl
from jax.experimental.pallas import tpu as pltpu
from jax.experimental.pallas import tpu_sc as plsc

_B = 16384
_D = 128
_T = 1000

_NC = 2
_NS = 16
_NW = _NC * _NS
_BPW = _B // _NW


def _sc_gather_body(t_hbm, la_hbm, lo_hbm, la_out_hbm, lo_out_hbm,
                    la_tab, lo_tab, idx_v, la_v, lo_v,
                    sem_a, sem_b, sem_c):
    wid = lax.axis_index("s") * _NC + lax.axis_index("c")
    base = wid * _BPW
    cp_a = pltpu.async_copy(la_hbm, la_tab, sem_a)
    cp_b = pltpu.async_copy(lo_hbm, lo_tab, sem_b)
    cp_c = pltpu.async_copy(t_hbm.at[pl.ds(base, _BPW)], idx_v, sem_c)
    cp_a.wait()
    cp_b.wait()
    cp_c.wait()
    for i in range(_BPW // 16):
        sl = pl.ds(i * 16, 16)
        idx = idx_v[sl]
        la_v[sl] = plsc.load_gather(la_tab, [idx])
        lo_v[sl] = plsc.load_gather(lo_tab, [idx])
    cp_d = pltpu.async_copy(la_v, la_out_hbm.at[pl.ds(base, _BPW)], sem_a)
    cp_e = pltpu.async_copy(lo_v, lo_out_hbm.at[pl.ds(base, _BPW)], sem_b)
    cp_d.wait()
    cp_e.wait()


_sc_gather = functools.partial(
    pl.kernel,
    mesh=plsc.VectorSubcoreMesh(core_axis_name="c", subcore_axis_name="s"),
    compiler_params=pltpu.CompilerParams(needs_layout_passes=False),
    out_type=(
        jax.ShapeDtypeStruct((_B,), jnp.float32),
        jax.ShapeDtypeStruct((_B,), jnp.float32),
    ),
    scratch_types=[
        pltpu.VMEM((_T,), jnp.float32),
        pltpu.VMEM((_T,), jnp.float32),
        pltpu.VMEM((_BPW,), jnp.int32),
        pltpu.VMEM((_BPW,), jnp.float32),
        pltpu.VMEM((_BPW,), jnp.float32),
        pltpu.SemaphoreType.DMA,
        pltpu.SemaphoreType.DMA,
        pltpu.SemaphoreType.DMA,
    ],
)(_sc_gather_body)


def _expand_rows(blk):
    r_sub = blk.shape[0]
    c = blk.reshape(r_sub, 1, _D)
    c = jnp.swapaxes(c, 1, 2)
    return c.reshape(r_sub * _D, 1)


def _tc_body(x_ref, la_ref, lo_ref, nc_ref, out_ref):
    la = _expand_rows(la_ref[...])
    lo = _expand_rows(lo_ref[...])
    a = la + x_ref[...]
    b = lo - jnp.log(nc_ref[...])
    m = jnp.maximum(a, b)
    d = jnp.minimum(a, b) - m
    out_ref[...] = m + jnp.log1p(jnp.exp(d))


_ROWS = 4096


def _tc_dense(log_x_0, la_t, lo_t, nc):
    grid = (_B // _ROWS,)
    return pl.pallas_call(
        _tc_body,
        grid=grid,
        in_specs=[
            pl.BlockSpec((_ROWS, _D), lambda i: (i, 0)),
            pl.BlockSpec((_ROWS // _D, _D), lambda i: (i, 0)),
            pl.BlockSpec((_ROWS // _D, _D), lambda i: (i, 0)),
            pl.BlockSpec((1, _D), lambda i: (0, 0)),
        ],
        out_specs=pl.BlockSpec((_ROWS, _D), lambda i: (i, 0)),
        out_shape=jax.ShapeDtypeStruct((_B, _D), jnp.float32),
    )(log_x_0, la_t, lo_t, nc)


def kernel(log_x_0, t, log_alpha_bar, log_one_minus_alpha_bar,
           num_classe_extended):
    la_t, lo_t = _sc_gather(t, log_alpha_bar, log_one_minus_alpha_bar)
    return _tc_dense(
        log_x_0,
        la_t.reshape(_B // _D, _D),
        lo_t.reshape(_B // _D, _D),
        num_classe_extended.reshape(1, _D),
    )


# speedup vs baseline: 1.2856x; 1.0071x over previous
import functools

import jax
import jax.numpy as jnp
from jax import lax
from jax.experimental import pallas as pl
from jax.experimental.pallas import tpu as pltpu
from jax.experimental.pallas import tpu_sc as plsc

_B = 16384
_D = 128
_T = 1000

_NC = 2
_NS = 16
_NW = _NC * _NS
_BPW = _B // _NW


def _sc_gather_body(t_hbm, la_hbm, lo_hbm, la_out_hbm, lo_out_hbm,
                    la_tab, lo_tab, idx_v, la_v, lo_v,
                    sem_a, sem_b, sem_c):
    wid = lax.axis_index("s") * _NC + lax.axis_index("c")
    base = wid * _BPW
    cp_a = pltpu.async_copy(la_hbm, la_tab, sem_a)
    cp_b = pltpu.async_copy(lo_hbm, lo_tab, sem_b)
    cp_c = pltpu.async_copy(t_hbm.at[pl.ds(base, _BPW)], idx_v, sem_c)
    cp_a.wait()
    cp_b.wait()
    cp_c.wait()
    def step(i, carry):
        sl = pl.ds(i * 16, 16)
        idx = idx_v[sl]
        la_v[sl] = plsc.load_gather(la_tab, [idx])
        lo_v[sl] = plsc.load_gather(lo_tab, [idx])
        return carry

    lax.fori_loop(0, _BPW // 16, step, 0)
    cp_d = pltpu.async_copy(la_v, la_out_hbm.at[pl.ds(base, _BPW)], sem_a)
    cp_e = pltpu.async_copy(lo_v, lo_out_hbm.at[pl.ds(base, _BPW)], sem_b)
    cp_d.wait()
    cp_e.wait()


_sc_gather = functools.partial(
    pl.kernel,
    mesh=plsc.VectorSubcoreMesh(core_axis_name="c", subcore_axis_name="s"),
    compiler_params=pltpu.CompilerParams(needs_layout_passes=False),
    out_type=(
        jax.ShapeDtypeStruct((_B,), jnp.float32),
        jax.ShapeDtypeStruct((_B,), jnp.float32),
    ),
    scratch_types=[
        pltpu.VMEM((_T,), jnp.float32),
        pltpu.VMEM((_T,), jnp.float32),
        pltpu.VMEM((_BPW,), jnp.int32),
        pltpu.VMEM((_BPW,), jnp.float32),
        pltpu.VMEM((_BPW,), jnp.float32),
        pltpu.SemaphoreType.DMA,
        pltpu.SemaphoreType.DMA,
        pltpu.SemaphoreType.DMA,
    ],
)(_sc_gather_body)


def _expand_rows(blk):
    r_sub = blk.shape[0]
    c = blk.reshape(r_sub, 1, _D)
    c = jnp.swapaxes(c, 1, 2)
    return c.reshape(r_sub * _D, 1)


def _tc_body(x_ref, la_ref, lo_ref, nc_ref, out_ref):
    la = _expand_rows(la_ref[...])
    lo = _expand_rows(lo_ref[...])
    a = la + x_ref[...]
    b = lo - jnp.log(nc_ref[...])
    m = jnp.maximum(a, b)
    d = jnp.minimum(a, b) - m
    out_ref[...] = m + jnp.log1p(jnp.exp(d))


_ROWS = 4096


def _tc_dense(log_x_0, la_t, lo_t, nc):
    grid = (_B // _ROWS,)
    return pl.pallas_call(
        _tc_body,
        grid=grid,
        in_specs=[
            pl.BlockSpec((_ROWS, _D), lambda i: (i, 0)),
            pl.BlockSpec((_ROWS // _D, _D), lambda i: (i, 0)),
            pl.BlockSpec((_ROWS // _D, _D), lambda i: (i, 0)),
            pl.BlockSpec((1, _D), lambda i: (0, 0)),
        ],
        out_specs=pl.BlockSpec((_ROWS, _D), lambda i: (i, 0)),
        out_shape=jax.ShapeDtypeStruct((_B, _D), jnp.float32),
    )(log_x_0, la_t, lo_t, nc)


def kernel(log_x_0, t, log_alpha_bar, log_one_minus_alpha_bar,
           num_classe_extended):
    la_t, lo_t = _sc_gather(t, log_alpha_bar, log_one_minus_alpha_bar)
    return _tc_dense(
        log_x_0,
        la_t.reshape(_B // _D, _D),
        lo_t.reshape(_B // _D, _D),
        num_classe_extended.reshape(1, _D),
    )


# speedup vs baseline: 1.3004x; 1.0115x over previous
import functools

import jax
import jax.numpy as jnp
from jax import lax
from jax.experimental import pallas as pl
from jax.experimental.pallas import tpu as pltpu
from jax.experimental.pallas import tpu_sc as plsc

_B = 16384
_D = 128
_T = 1000

_NC = 2
_NS = 16
_NW = _NC * _NS
_BPW = _B // _NW


def _sc_gather_body(t_hbm, la_hbm, lo_hbm, la_out_hbm, lo_out_hbm,
                    la_tab, lo_tab, idx_v, la_v, lo_v,
                    sem_a, sem_b, sem_c):
    wid = lax.axis_index("s") * _NC + lax.axis_index("c")
    base = wid * _BPW
    cp_a = pltpu.async_copy(la_hbm, la_tab, sem_a)
    cp_b = pltpu.async_copy(lo_hbm, lo_tab, sem_b)
    cp_c = pltpu.async_copy(t_hbm.at[pl.ds(base, _BPW)], idx_v, sem_c)
    cp_a.wait()
    cp_b.wait()
    cp_c.wait()
    def step(i, carry):
        sl = pl.ds(i * 16, 16)
        idx = idx_v[sl]
        la_v[sl] = plsc.load_gather(la_tab, [idx])
        lo_v[sl] = plsc.load_gather(lo_tab, [idx])
        return carry

    lax.fori_loop(0, _BPW // 16, step, 0)
    cp_d = pltpu.async_copy(la_v, la_out_hbm.at[pl.ds(base, _BPW)], sem_a)
    cp_e = pltpu.async_copy(lo_v, lo_out_hbm.at[pl.ds(base, _BPW)], sem_b)
    cp_d.wait()
    cp_e.wait()


_sc_gather = functools.partial(
    pl.kernel,
    mesh=plsc.VectorSubcoreMesh(core_axis_name="c", subcore_axis_name="s"),
    compiler_params=pltpu.CompilerParams(needs_layout_passes=False),
    out_type=(
        jax.ShapeDtypeStruct((_B,), jnp.float32),
        jax.ShapeDtypeStruct((_B,), jnp.float32),
    ),
    scratch_types=[
        pltpu.VMEM((_T,), jnp.float32),
        pltpu.VMEM((_T,), jnp.float32),
        pltpu.VMEM((_BPW,), jnp.int32),
        pltpu.VMEM((_BPW,), jnp.float32),
        pltpu.VMEM((_BPW,), jnp.float32),
        pltpu.SemaphoreType.DMA,
        pltpu.SemaphoreType.DMA,
        pltpu.SemaphoreType.DMA,
    ],
)(_sc_gather_body)


def _expand_rows(blk):
    r_sub = blk.shape[0]
    c = blk.reshape(r_sub, 1, _D)
    c = jnp.swapaxes(c, 1, 2)
    return c.reshape(r_sub * _D, 1)


def _tc_body(x_ref, la_ref, lo_ref, nc_ref, out_ref):
    la = _expand_rows(la_ref[...])
    lo = _expand_rows(lo_ref[...])
    a = la + x_ref[...]
    b = lo - jnp.log(nc_ref[...])
    m = jnp.maximum(a, b)
    z = jnp.exp2(jnp.abs(a - b) * (-1.4426950408889634))
    out_ref[...] = m + jnp.log2(1.0 + z) * 0.6931471805599453


_ROWS = 2048


def _tc_dense(log_x_0, la_t, lo_t, nc):
    grid = (_B // _ROWS,)
    return pl.pallas_call(
        _tc_body,
        grid=grid,
        in_specs=[
            pl.BlockSpec((_ROWS, _D), lambda i: (i, 0)),
            pl.BlockSpec((_ROWS // _D, _D), lambda i: (i, 0)),
            pl.BlockSpec((_ROWS // _D, _D), lambda i: (i, 0)),
            pl.BlockSpec((1, _D), lambda i: (0, 0)),
        ],
        out_specs=pl.BlockSpec((_ROWS, _D), lambda i: (i, 0)),
        out_shape=jax.ShapeDtypeStruct((_B, _D), jnp.float32),
    )(log_x_0, la_t, lo_t, nc)


def kernel(log_x_0, t, log_alpha_bar, log_one_minus_alpha_bar,
           num_classe_extended):
    la_t, lo_t = _sc_gather(t, log_alpha_bar, log_one_minus_alpha_bar)
    return _tc_dense(
        log_x_0,
        la_t.reshape(_B // _D, _D),
        lo_t.reshape(_B // _D, _D),
        num_classe_extended.reshape(1, _D),
    )
